# baseline (speedup 1.0000x reference)
import numpy as np
import jax
import jax.numpy as jnp
from jax.experimental import pallas as pl
from jax.experimental.pallas import tpu as pltpu


def _net_kernel(x_ref, t1e_ref, t1o_ref, b1_ref, g2e_ref, g2o_ref, b2_ref,
                wf1_ref, bf1_ref, wf2_ref, bf2_ref, wf3_ref, bf3_ref,
                out_ref, p2a_ref, p2b_ref):
    BB = x_ref.shape[0] // 15
    S = BB * 15 - 1

    def dotf(a, b):
        return jnp.dot(a, b, preferred_element_type=jnp.float32)

    x4 = x_ref[...]
    a0 = x4[0:S, :]
    a1 = x4[1:S + 1, :]
    op0 = jnp.concatenate([a0, a1[:, 0:60]], axis=1)
    op1 = jnp.concatenate([a0[:, 60:240], a1[:, 0:120]], axis=1)
    op2 = jnp.concatenate([a0[:, 120:240], a1[:, 0:180]], axis=1)
    op3 = jnp.concatenate([a0[:, 180:240], a1], axis=1)
    t1e = t1e_ref[...]
    t1o = t1o_ref[...]
    b1 = b1_ref[...]
    p1_lo = jnp.maximum(jnp.maximum(
        jnp.maximum(dotf(op0, t1e), dotf(op0, t1o)),
        jnp.maximum(dotf(op1, t1e), dotf(op1, t1o))) + b1,
        0.0).astype(jnp.bfloat16)
    p1_hi = jnp.maximum(jnp.maximum(
        jnp.maximum(dotf(op2, t1e), dotf(op2, t1o)),
        jnp.maximum(dotf(op3, t1e), dotf(op3, t1o))) + b1,
        0.0).astype(jnp.bfloat16)

    n2 = BB * 15 - 3
    lo = lambda k: p1_lo[k:k + n2, :]
    hi = lambda k: p1_hi[k:k + n2, :]
    oe = jnp.concatenate([lo(0), hi(0), lo(1), hi(1), lo(2)], axis=1)
    oo = jnp.concatenate([hi(0), lo(1), hi(1), lo(2), hi(2)], axis=1)
    g2e = g2e_ref[...]
    g2o = g2o_ref[...]
    p2 = jnp.maximum(jnp.maximum(
        jnp.maximum(dotf(oe, g2e), dotf(oe, g2o)),
        jnp.maximum(dotf(oo, g2e), dotf(oo, g2o))) + b2_ref[...],
        0.0)
    p2a_ref[0:n2, :] = p2[:, 0:128]
    p2b_ref[0:n2, 0:64] = p2[:, 128:192]
    p2a_ref[n2:, :] = jnp.zeros((p2a_ref.shape[0] - n2, 128), jnp.float32)
    p2b_ref[n2:, 0:64] = jnp.zeros((p2b_ref.shape[0] - n2, 64), jnp.float32)

    wf1 = wf1_ref[...]

    def p2_rows(y):
        return jnp.concatenate(
            [p2a_ref[pl.dslice(y, BB, 15), :],
             p2b_ref[pl.dslice(y, BB, 15), 0:64]], axis=1).astype(jnp.bfloat16)

    h = jnp.dot(p2_rows(0), wf1[0:192, :], preferred_element_type=jnp.float32)
    for y in range(1, 12):
        h = h + jnp.dot(p2_rows(y), wf1[y * 192:(y + 1) * 192, :],
                        preferred_element_type=jnp.float32)
    h = jnp.maximum(h + bf1_ref[...], 0.0).astype(jnp.bfloat16)
    h = jnp.maximum(
        jnp.dot(h, wf2_ref[...], preferred_element_type=jnp.float32)
        + bf2_ref[...], 0.0).astype(jnp.bfloat16)
    out_ref[...] = (jnp.dot(h, wf3_ref[...], preferred_element_type=jnp.float32)
                    + bf3_ref[...])


def _absorbed_operators(w1, w2):
    s1 = np.stack([np.eye(60, 56, k=-d) for d in range(5)]).astype(np.float32)
    t1 = jnp.einsum('dpj,cyd->ypcj', s1, w1[:, 0]).reshape(300, 336)
    s2 = np.stack([np.eye(28, 24, k=-d) for d in range(5)]).astype(np.float32)
    g2 = jnp.einsum('dmx,ocyd->ycmox', s2, w2).reshape(840, 384)
    i1 = np.repeat(np.arange(6), 28) * 56 + 2 * np.tile(np.arange(28), 6)
    i2 = np.repeat(np.arange(16), 12) * 24 + 2 * np.tile(np.arange(12), 16)
    return t1[:, i1], t1[:, i1 + 1], g2[:, i2], g2[:, i2 + 1]


def _pick_block(B, candidates):
    for c in candidates:
        if B % c == 0:
            return c
    return 1


def kernel(x, w1, b1, w2, b2, wf1, bf1, wf2, bf2, wf3, bf3):
    B = x.shape[0]
    BB = _pick_block(B, (64, 32, 16, 8, 4, 2))
    nb = B // BB

    t1e, t1o, g2e, g2o = [a.astype(jnp.bfloat16)
                          for a in _absorbed_operators(w1, w2)]
    b1_row = jnp.repeat(b1, 28)[None, :]
    b2_row = jnp.repeat(b2, 12)[None, :]
    wf1_perm = wf1.reshape(16, 12, 12, 120).transpose(1, 0, 2, 3)
    wf1_perm = wf1_perm.reshape(2304, 120)

    full = lambda a: pl.BlockSpec(a.shape, lambda b: tuple(0 for _ in a.shape))
    row = lambda n: pl.BlockSpec((1, n), lambda b: (0, 0))
    return pl.pallas_call(
        _net_kernel,
        grid=(nb,),
        in_specs=[
            pl.BlockSpec((BB * 15, 240), lambda b: (b, 0)),
            full(t1e), full(t1o), row(168),
            full(g2e), full(g2o), row(192),
            full(wf1_perm), row(120), full(wf2), row(84), full(wf3), row(2),
        ],
        out_specs=pl.BlockSpec((BB, 2), lambda b: (b, 0)),
        out_shape=jax.ShapeDtypeStruct((B, 2), jnp.float32),
        scratch_shapes=[
            pltpu.VMEM((BB * 15, 128), jnp.float32),
            pltpu.VMEM((BB * 15, 128), jnp.float32),
        ],
        compiler_params=pltpu.CompilerParams(
            dimension_semantics=("parallel",)),
    )(x.astype(jnp.bfloat16).reshape(B * 15, 240), t1e, t1o, b1_row,
      g2e, g2o, b2_row,
      wf1_perm.astype(jnp.bfloat16), bf1.reshape(1, 120),
      wf2.astype(jnp.bfloat16), bf2.reshape(1, 84),
      wf3.astype(jnp.bfloat16), bf3.reshape(1, 2))


# speedup vs baseline: 5.6770x; 5.6770x over previous
import numpy as np
import jax
import jax.numpy as jnp
from jax.experimental import pallas as pl
from jax.experimental.pallas import tpu as pltpu


def _repack_kernel(x_ref, o_ref):
    x3 = x_ref[:, 0, :, :]
    for u in range(15):
        o_ref[:, u, :] = jnp.concatenate(
            [x3[:, 4 * u + q, :] for q in range(4)],
            axis=1).astype(jnp.bfloat16)


def _repack(x, RB):
    B = x.shape[0]
    return pl.pallas_call(
        _repack_kernel,
        grid=(B // RB,),
        in_specs=[pl.BlockSpec((RB, 1, 60, 60), lambda b: (b, 0, 0, 0))],
        out_specs=pl.BlockSpec((RB, 15, 240), lambda b: (b, 0, 0)),
        out_shape=jax.ShapeDtypeStruct((B, 15, 240), jnp.bfloat16),
        compiler_params=pltpu.CompilerParams(
            dimension_semantics=("parallel",)),
    )(x)


def _net_kernel(x_ref, t1e_ref, t1o_ref, b1_ref, g2e_ref, g2o_ref, b2_ref,
                wf1_ref, bf1_ref, wf2_ref, bf2_ref, wf3_ref, bf3_ref,
                out_ref, p2a_ref, p2b_ref):
    BB = x_ref.shape[0] // 15
    S = BB * 15 - 1

    def dotf(a, b):
        return jnp.dot(a, b, preferred_element_type=jnp.float32)

    x4 = x_ref[...]
    a0 = x4[0:S, :]
    a1 = x4[1:S + 1, :]
    op0 = jnp.concatenate([a0, a1[:, 0:60]], axis=1)
    op1 = jnp.concatenate([a0[:, 60:240], a1[:, 0:120]], axis=1)
    op2 = jnp.concatenate([a0[:, 120:240], a1[:, 0:180]], axis=1)
    op3 = jnp.concatenate([a0[:, 180:240], a1], axis=1)
    t1e = t1e_ref[...]
    t1o = t1o_ref[...]
    b1 = b1_ref[...]
    p1_lo = jnp.maximum(jnp.maximum(
        jnp.maximum(dotf(op0, t1e), dotf(op0, t1o)),
        jnp.maximum(dotf(op1, t1e), dotf(op1, t1o))) + b1,
        0.0).astype(jnp.bfloat16)
    p1_hi = jnp.maximum(jnp.maximum(
        jnp.maximum(dotf(op2, t1e), dotf(op2, t1o)),
        jnp.maximum(dotf(op3, t1e), dotf(op3, t1o))) + b1,
        0.0).astype(jnp.bfloat16)

    n2 = BB * 15 - 3
    lo = lambda k: p1_lo[k:k + n2, :]
    hi = lambda k: p1_hi[k:k + n2, :]
    oe = jnp.concatenate([lo(0), hi(0), lo(1), hi(1), lo(2)], axis=1)
    oo = jnp.concatenate([hi(0), lo(1), hi(1), lo(2), hi(2)], axis=1)
    g2e = g2e_ref[...]
    g2o = g2o_ref[...]
    p2 = jnp.maximum(jnp.maximum(
        jnp.maximum(dotf(oe, g2e), dotf(oe, g2o)),
        jnp.maximum(dotf(oo, g2e), dotf(oo, g2o))) + b2_ref[...],
        0.0)
    p2a_ref[0:n2, :] = p2[:, 0:128]
    p2b_ref[0:n2, 0:64] = p2[:, 128:192]
    p2a_ref[n2:, :] = jnp.zeros((p2a_ref.shape[0] - n2, 128), jnp.float32)
    p2b_ref[n2:, 0:64] = jnp.zeros((p2b_ref.shape[0] - n2, 64), jnp.float32)

    wf1 = wf1_ref[...]

    def p2_rows(y):
        return jnp.concatenate(
            [p2a_ref[pl.dslice(y, BB, 15), :],
             p2b_ref[pl.dslice(y, BB, 15), 0:64]], axis=1).astype(jnp.bfloat16)

    h = jnp.dot(p2_rows(0), wf1[0:192, :], preferred_element_type=jnp.float32)
    for y in range(1, 12):
        h = h + jnp.dot(p2_rows(y), wf1[y * 192:(y + 1) * 192, :],
                        preferred_element_type=jnp.float32)
    h = jnp.maximum(h + bf1_ref[...], 0.0).astype(jnp.bfloat16)
    h = jnp.maximum(
        jnp.dot(h, wf2_ref[...], preferred_element_type=jnp.float32)
        + bf2_ref[...], 0.0).astype(jnp.bfloat16)
    out_ref[...] = (jnp.dot(h, wf3_ref[...], preferred_element_type=jnp.float32)
                    + bf3_ref[...])


def _absorbed_operators(w1, w2):
    s1 = np.stack([np.eye(60, 56, k=-d) for d in range(5)]).astype(np.float32)
    t1 = jnp.einsum('dpj,cyd->ypcj', s1, w1[:, 0]).reshape(300, 336)
    s2 = np.stack([np.eye(28, 24, k=-d) for d in range(5)]).astype(np.float32)
    g2 = jnp.einsum('dmx,ocyd->ycmox', s2, w2).reshape(840, 384)
    i1 = np.repeat(np.arange(6), 28) * 56 + 2 * np.tile(np.arange(28), 6)
    i2 = np.repeat(np.arange(16), 12) * 24 + 2 * np.tile(np.arange(12), 16)
    return t1[:, i1], t1[:, i1 + 1], g2[:, i2], g2[:, i2 + 1]


def _pick_block(B, candidates):
    for c in candidates:
        if B % c == 0:
            return c
    return 1


def kernel(x, w1, b1, w2, b2, wf1, bf1, wf2, bf2, wf3, bf3):
    B = x.shape[0]
    BB = _pick_block(B, (64, 32, 16, 8, 4, 2))
    nb = B // BB

    t1e, t1o, g2e, g2o = [a.astype(jnp.bfloat16)
                          for a in _absorbed_operators(w1, w2)]
    b1_row = jnp.repeat(b1, 28)[None, :]
    b2_row = jnp.repeat(b2, 12)[None, :]
    wf1_perm = wf1.reshape(16, 12, 12, 120).transpose(1, 0, 2, 3)
    wf1_perm = wf1_perm.reshape(2304, 120)

    full = lambda a: pl.BlockSpec(a.shape, lambda b: tuple(0 for _ in a.shape))
    row = lambda n: pl.BlockSpec((1, n), lambda b: (0, 0))
    return pl.pallas_call(
        _net_kernel,
        grid=(nb,),
        in_specs=[
            pl.BlockSpec((BB * 15, 240), lambda b: (b, 0)),
            full(t1e), full(t1o), row(168),
            full(g2e), full(g2o), row(192),
            full(wf1_perm), row(120), full(wf2), row(84), full(wf3), row(2),
        ],
        out_specs=pl.BlockSpec((BB, 2), lambda b: (b, 0)),
        out_shape=jax.ShapeDtypeStruct((B, 2), jnp.float32),
        scratch_shapes=[
            pltpu.VMEM((BB * 15, 128), jnp.float32),
            pltpu.VMEM((BB * 15, 128), jnp.float32),
        ],
        compiler_params=pltpu.CompilerParams(
            dimension_semantics=("parallel",)),
    )(_repack(x, _pick_block(B, (128, 64, 32, 16, 8, 4, 2))
               ).reshape(B * 15, 240), t1e, t1o, b1_row,
      g2e, g2o, b2_row,
      wf1_perm.astype(jnp.bfloat16), bf1.reshape(1, 120),
      wf2.astype(jnp.bfloat16), bf2.reshape(1, 84),
      wf3.astype(jnp.bfloat16), bf3.reshape(1, 2))


# speedup vs baseline: 8.3563x; 1.4720x over previous
import numpy as np
import jax
import jax.numpy as jnp
from jax.experimental import pallas as pl
from jax.experimental.pallas import tpu as pltpu


def _net_kernel(x_ref, t1e_ref, t1o_ref, b1_ref, g2e_ref, g2o_ref, b2_ref,
                wf1_ref, bf1_ref, wf2_ref, bf2_ref, wf3_ref, bf3_ref,
                out_ref, p2a_ref, p2b_ref):
    BB = x_ref.shape[0] // 15
    S = BB * 15 - 1

    def dotf(a, b):
        return jnp.dot(a, b, preferred_element_type=jnp.float32)

    x4 = x_ref[...].astype(jnp.bfloat16)
    a0 = x4[0:S, :]
    a1 = x4[1:S + 1, :]
    op0 = jnp.concatenate([a0, a1[:, 0:60]], axis=1)
    op1 = jnp.concatenate([a0[:, 60:240], a1[:, 0:120]], axis=1)
    op2 = jnp.concatenate([a0[:, 120:240], a1[:, 0:180]], axis=1)
    op3 = jnp.concatenate([a0[:, 180:240], a1], axis=1)
    t1e = t1e_ref[...]
    t1o = t1o_ref[...]
    b1 = b1_ref[...]
    p1_lo = jnp.maximum(jnp.maximum(
        jnp.maximum(dotf(op0, t1e), dotf(op0, t1o)),
        jnp.maximum(dotf(op1, t1e), dotf(op1, t1o))) + b1,
        0.0).astype(jnp.bfloat16)
    p1_hi = jnp.maximum(jnp.maximum(
        jnp.maximum(dotf(op2, t1e), dotf(op2, t1o)),
        jnp.maximum(dotf(op3, t1e), dotf(op3, t1o))) + b1,
        0.0).astype(jnp.bfloat16)

    n2 = BB * 15 - 3
    lo = lambda k: p1_lo[k:k + n2, :]
    hi = lambda k: p1_hi[k:k + n2, :]
    oe = jnp.concatenate([lo(0), hi(0), lo(1), hi(1), lo(2)], axis=1)
    oo = jnp.concatenate([hi(0), lo(1), hi(1), lo(2), hi(2)], axis=1)
    g2e = g2e_ref[...]
    g2o = g2o_ref[...]
    p2 = jnp.maximum(jnp.maximum(
        jnp.maximum(dotf(oe, g2e), dotf(oe, g2o)),
        jnp.maximum(dotf(oo, g2e), dotf(oo, g2o))) + b2_ref[...],
        0.0)
    p2a_ref[0:n2, :] = p2[:, 0:128]
    p2b_ref[0:n2, 0:64] = p2[:, 128:192]
    p2a_ref[n2:, :] = jnp.zeros((p2a_ref.shape[0] - n2, 128), jnp.float32)
    p2b_ref[n2:, 0:64] = jnp.zeros((p2b_ref.shape[0] - n2, 64), jnp.float32)

    wf1 = wf1_ref[...]

    def p2_rows(y):
        return jnp.concatenate(
            [p2a_ref[pl.dslice(y, BB, 15), :],
             p2b_ref[pl.dslice(y, BB, 15), 0:64]], axis=1).astype(jnp.bfloat16)

    h = jnp.dot(p2_rows(0), wf1[0:192, :], preferred_element_type=jnp.float32)
    for y in range(1, 12):
        h = h + jnp.dot(p2_rows(y), wf1[y * 192:(y + 1) * 192, :],
                        preferred_element_type=jnp.float32)
    h = jnp.maximum(h + bf1_ref[...], 0.0).astype(jnp.bfloat16)
    h = jnp.maximum(
        jnp.dot(h, wf2_ref[...], preferred_element_type=jnp.float32)
        + bf2_ref[...], 0.0).astype(jnp.bfloat16)
    out_ref[...] = (jnp.dot(h, wf3_ref[...], preferred_element_type=jnp.float32)
                    + bf3_ref[...])


def _absorbed_operators(w1, w2):
    s1 = np.stack([np.eye(60, 56, k=-d) for d in range(5)]).astype(np.float32)
    t1 = jnp.einsum('dpj,cyd->ypcj', s1, w1[:, 0]).reshape(300, 336)
    s2 = np.stack([np.eye(28, 24, k=-d) for d in range(5)]).astype(np.float32)
    g2 = jnp.einsum('dmx,ocyd->ycmox', s2, w2).reshape(840, 384)
    i1 = np.repeat(np.arange(6), 28) * 56 + 2 * np.tile(np.arange(28), 6)
    i2 = np.repeat(np.arange(16), 12) * 24 + 2 * np.tile(np.arange(12), 16)
    return t1[:, i1], t1[:, i1 + 1], g2[:, i2], g2[:, i2 + 1]


def _pick_block(B, candidates):
    for c in candidates:
        if B % c == 0:
            return c
    return 1


def kernel(x, w1, b1, w2, b2, wf1, bf1, wf2, bf2, wf3, bf3):
    B = x.shape[0]
    BB = _pick_block(B, (64, 32, 16, 8, 4, 2))
    nb = B // BB

    t1e, t1o, g2e, g2o = [a.astype(jnp.bfloat16)
                          for a in _absorbed_operators(w1, w2)]
    b1_row = jnp.repeat(b1, 28)[None, :]
    b2_row = jnp.repeat(b2, 12)[None, :]
    wf1_perm = wf1.reshape(16, 12, 12, 120).transpose(1, 0, 2, 3)
    wf1_perm = wf1_perm.reshape(2304, 120)

    full = lambda a: pl.BlockSpec(a.shape, lambda b: tuple(0 for _ in a.shape))
    row = lambda n: pl.BlockSpec((1, n), lambda b: (0, 0))
    return pl.pallas_call(
        _net_kernel,
        grid=(nb,),
        in_specs=[
            pl.BlockSpec((BB * 15, 240), lambda b: (b, 0)),
            full(t1e), full(t1o), row(168),
            full(g2e), full(g2o), row(192),
            full(wf1_perm), row(120), full(wf2), row(84), full(wf3), row(2),
        ],
        out_specs=pl.BlockSpec((BB, 2), lambda b: (b, 0)),
        out_shape=jax.ShapeDtypeStruct((B, 2), jnp.float32),
        scratch_shapes=[
            pltpu.VMEM((BB * 15, 128), jnp.float32),
            pltpu.VMEM((BB * 15, 128), jnp.float32),
        ],
        compiler_params=pltpu.CompilerParams(
            dimension_semantics=("parallel",)),
    )(x.reshape(B * 15, 240), t1e, t1o, b1_row,
      g2e, g2o, b2_row,
      wf1_perm.astype(jnp.bfloat16), bf1.reshape(1, 120),
      wf2.astype(jnp.bfloat16), bf2.reshape(1, 84),
      wf3.astype(jnp.bfloat16), bf3.reshape(1, 2))


# speedup vs baseline: 10.4462x; 1.2501x over previous
import numpy as np
import jax
import jax.numpy as jnp
from jax.experimental import pallas as pl
from jax.experimental.pallas import tpu as pltpu


def _net_kernel(x_ref, t1e_ref, t1o_ref, b1_ref, g2e_ref, g2o_ref, b2_ref,
                wf1_ref, bf1_ref, wf2_ref, bf2_ref, wf3_ref, bf3_ref,
                out_ref, ma_ref, mb_ref, p2a_ref, p2b_ref):
    BB = x_ref.shape[0]

    def dotf(a, b):
        return jnp.dot(a, b, preferred_element_type=jnp.float32)

    x3 = x_ref[...].astype(jnp.bfloat16)
    xr = jnp.concatenate([x3[:, d:d + 56, :] for d in range(5)],
                         axis=2).reshape(BB * 56, 300)
    m = jnp.maximum(dotf(xr, t1e_ref[...]),
                    dotf(xr, t1o_ref[...]))
    ma_ref[...] = m[:, 0:128]
    mb_ref[:, 0:40] = m[:, 128:168]

    n1 = BB * 14

    def mrow(q):
        return jnp.concatenate(
            [ma_ref[pl.dslice(q, n1, 4), :],
             mb_ref[pl.dslice(q, n1, 4), 0:40]], axis=1)

    b1 = b1_ref[...]
    p1_lo = jnp.maximum(jnp.maximum(mrow(0), mrow(1)) + b1,
                        0.0).astype(jnp.bfloat16)
    p1_hi = jnp.maximum(jnp.maximum(mrow(2), mrow(3)) + b1,
                        0.0).astype(jnp.bfloat16)

    n2 = BB * 14 - 2
    lo = lambda k: p1_lo[k:k + n2, :]
    hi = lambda k: p1_hi[k:k + n2, :]
    oe = jnp.concatenate([lo(0), hi(0), lo(1), hi(1), lo(2)], axis=1)
    oo = jnp.concatenate([hi(0), lo(1), hi(1), lo(2), hi(2)], axis=1)
    g2e = g2e_ref[...]
    g2o = g2o_ref[...]
    p2 = jnp.maximum(jnp.maximum(
        jnp.maximum(dotf(oe, g2e), dotf(oe, g2o)),
        jnp.maximum(dotf(oo, g2e), dotf(oo, g2o))) + b2_ref[...],
        0.0)
    p2a_ref[0:n2, :] = p2[:, 0:128]
    p2b_ref[0:n2, 0:64] = p2[:, 128:192]
    p2a_ref[n2:, :] = jnp.zeros((p2a_ref.shape[0] - n2, 128), jnp.float32)
    p2b_ref[n2:, 0:64] = jnp.zeros((p2b_ref.shape[0] - n2, 64), jnp.float32)

    wf1 = wf1_ref[...]

    def p2_rows(y):
        return jnp.concatenate(
            [p2a_ref[pl.dslice(y, BB, 14), :],
             p2b_ref[pl.dslice(y, BB, 14), 0:64]], axis=1).astype(jnp.bfloat16)

    h = jnp.dot(p2_rows(0), wf1[0:192, :], preferred_element_type=jnp.float32)
    for y in range(1, 12):
        h = h + jnp.dot(p2_rows(y), wf1[y * 192:(y + 1) * 192, :],
                        preferred_element_type=jnp.float32)
    h = jnp.maximum(h + bf1_ref[...], 0.0).astype(jnp.bfloat16)
    h = jnp.maximum(
        jnp.dot(h, wf2_ref[...], preferred_element_type=jnp.float32)
        + bf2_ref[...], 0.0).astype(jnp.bfloat16)
    out_ref[...] = (jnp.dot(h, wf3_ref[...], preferred_element_type=jnp.float32)
                    + bf3_ref[...])


def _absorbed_operators(w1, w2):
    s1 = np.stack([np.eye(60, 56, k=-d) for d in range(5)]).astype(np.float32)
    t1 = jnp.einsum('dpj,cyd->ypcj', s1, w1[:, 0]).reshape(300, 336)
    s2 = np.stack([np.eye(28, 24, k=-d) for d in range(5)]).astype(np.float32)
    g2 = jnp.einsum('dmx,ocyd->ycmox', s2, w2).reshape(840, 384)
    i1 = np.repeat(np.arange(6), 28) * 56 + 2 * np.tile(np.arange(28), 6)
    i2 = np.repeat(np.arange(16), 12) * 24 + 2 * np.tile(np.arange(12), 16)
    return t1[:, i1], t1[:, i1 + 1], g2[:, i2], g2[:, i2 + 1]


def _pick_block(B, candidates):
    for c in candidates:
        if B % c == 0:
            return c
    return 1


def kernel(x, w1, b1, w2, b2, wf1, bf1, wf2, bf2, wf3, bf3):
    B = x.shape[0]
    BB = _pick_block(B, (64, 32, 16, 8, 4, 2))
    nb = B // BB

    t1e, t1o, g2e, g2o = [a.astype(jnp.bfloat16)
                          for a in _absorbed_operators(w1, w2)]
    b1_row = jnp.repeat(b1, 28)[None, :]
    b2_row = jnp.repeat(b2, 12)[None, :]
    wf1_perm = wf1.reshape(16, 12, 12, 120).transpose(1, 0, 2, 3)
    wf1_perm = wf1_perm.reshape(2304, 120)

    full = lambda a: pl.BlockSpec(a.shape, lambda b: tuple(0 for _ in a.shape))
    row = lambda n: pl.BlockSpec((1, n), lambda b: (0, 0))
    return pl.pallas_call(
        _net_kernel,
        grid=(nb,),
        in_specs=[
            pl.BlockSpec((BB, 60, 60), lambda b: (b, 0, 0)),
            full(t1e), full(t1o), row(168),
            full(g2e), full(g2o), row(192),
            full(wf1_perm), row(120), full(wf2), row(84), full(wf3), row(2),
        ],
        out_specs=pl.BlockSpec((BB, 2), lambda b: (b, 0)),
        out_shape=jax.ShapeDtypeStruct((B, 2), jnp.float32),
        scratch_shapes=[
            pltpu.VMEM((BB * 56, 128), jnp.float32),
            pltpu.VMEM((BB * 56, 128), jnp.float32),
            pltpu.VMEM((BB * 14, 128), jnp.float32),
            pltpu.VMEM((BB * 14, 128), jnp.float32),
        ],
        compiler_params=pltpu.CompilerParams(
            dimension_semantics=("parallel",)),
    )(x.reshape(B, 60, 60), t1e, t1o, b1_row,
      g2e, g2o, b2_row,
      wf1_perm.astype(jnp.bfloat16), bf1.reshape(1, 120),
      wf2.astype(jnp.bfloat16), bf2.reshape(1, 84),
      wf3.astype(jnp.bfloat16), bf3.reshape(1, 2))


# speedup vs baseline: 11.0061x; 1.0536x over previous
import numpy as np
import jax
import jax.numpy as jnp
from jax.experimental import pallas as pl
from jax.experimental.pallas import tpu as pltpu


def _net_kernel(x_ref, t1e_ref, t1o_ref, b1_ref, g2e_ref, g2o_ref, b2_ref,
                wf1_ref, bf1_ref, wf2_ref, bf2_ref, wf3_ref, bf3_ref,
                out_ref, ma_ref, mb_ref, p2a_ref, p2b_ref):
    BB = x_ref.shape[0]

    def dotf(a, b):
        return jnp.dot(a, b, preferred_element_type=jnp.float32)

    x3 = x_ref[...].astype(jnp.bfloat16)
    xr = jnp.concatenate([x3[:, d:d + 56, :] for d in range(5)],
                         axis=2).reshape(BB * 56, 300)
    m = jnp.maximum(dotf(xr, t1e_ref[...]),
                    dotf(xr, t1o_ref[...]))
    ma_ref[...] = m[:, 0:128]
    mb_ref[:, 0:40] = m[:, 128:168]

    n1 = BB * 14

    def mrow(q):
        return jnp.concatenate(
            [ma_ref[pl.dslice(q, n1, 4), :],
             mb_ref[pl.dslice(q, n1, 4), 0:40]], axis=1)

    b1 = b1_ref[...]
    p1_lo = jnp.maximum(jnp.maximum(mrow(0), mrow(1)) + b1,
                        0.0).astype(jnp.bfloat16)
    p1_hi = jnp.maximum(jnp.maximum(mrow(2), mrow(3)) + b1,
                        0.0).astype(jnp.bfloat16)

    n2 = BB * 14 - 2
    lo = lambda k: p1_lo[k:k + n2, :]
    hi = lambda k: p1_hi[k:k + n2, :]
    oe = jnp.concatenate([lo(0), hi(0), lo(1), hi(1), lo(2)], axis=1)
    oo = jnp.concatenate([hi(0), lo(1), hi(1), lo(2), hi(2)], axis=1)
    g2e = g2e_ref[...]
    g2o = g2o_ref[...]
    p2 = jnp.maximum(jnp.maximum(
        jnp.maximum(dotf(oe, g2e), dotf(oe, g2o)),
        jnp.maximum(dotf(oo, g2e), dotf(oo, g2o))) + b2_ref[...],
        0.0)
    p2a_ref[0:n2, :] = p2[:, 0:128]
    p2b_ref[0:n2, 0:64] = p2[:, 128:192]
    p2a_ref[n2:, :] = jnp.zeros((p2a_ref.shape[0] - n2, 128), jnp.float32)
    p2b_ref[n2:, 0:64] = jnp.zeros((p2b_ref.shape[0] - n2, 64), jnp.float32)

    wf1 = wf1_ref[...]

    def p2_rows(y):
        return jnp.concatenate(
            [p2a_ref[pl.dslice(y, BB, 14), :],
             p2b_ref[pl.dslice(y, BB, 14), 0:64]], axis=1).astype(jnp.bfloat16)

    h = jnp.dot(p2_rows(0), wf1[0:192, :], preferred_element_type=jnp.float32)
    for y in range(1, 12):
        h = h + jnp.dot(p2_rows(y), wf1[y * 192:(y + 1) * 192, :],
                        preferred_element_type=jnp.float32)
    h = jnp.maximum(h + bf1_ref[...], 0.0).astype(jnp.bfloat16)
    h = jnp.maximum(
        jnp.dot(h, wf2_ref[...], preferred_element_type=jnp.float32)
        + bf2_ref[...], 0.0).astype(jnp.bfloat16)
    out_ref[...] = (jnp.dot(h, wf3_ref[...], preferred_element_type=jnp.float32)
                    + bf3_ref[...])


def _absorbed_operators(w1, w2):
    s1 = np.stack([np.eye(60, 56, k=-d) for d in range(5)]).astype(np.float32)
    t1 = jnp.einsum('dpj,cyd->ypcj', s1, w1[:, 0]).reshape(300, 336)
    s2 = np.stack([np.eye(28, 24, k=-d) for d in range(5)]).astype(np.float32)
    g2 = jnp.einsum('dmx,ocyd->ycmox', s2, w2).reshape(840, 384)
    i1 = np.repeat(np.arange(6), 28) * 56 + 2 * np.tile(np.arange(28), 6)
    i2 = np.repeat(np.arange(16), 12) * 24 + 2 * np.tile(np.arange(12), 16)
    return t1[:, i1], t1[:, i1 + 1], g2[:, i2], g2[:, i2 + 1]


def _pick_block(B, candidates):
    for c in candidates:
        if B % c == 0:
            return c
    return 1


def kernel(x, w1, b1, w2, b2, wf1, bf1, wf2, bf2, wf3, bf3):
    B = x.shape[0]
    BB = _pick_block(B, (128, 64, 32, 16, 8, 4, 2))
    nb = B // BB

    t1e, t1o, g2e, g2o = [a.astype(jnp.bfloat16)
                          for a in _absorbed_operators(w1, w2)]
    b1_row = jnp.repeat(b1, 28)[None, :]
    b2_row = jnp.repeat(b2, 12)[None, :]
    wf1_perm = wf1.reshape(16, 12, 12, 120).transpose(1, 0, 2, 3)
    wf1_perm = wf1_perm.reshape(2304, 120)

    full = lambda a: pl.BlockSpec(a.shape, lambda b: tuple(0 for _ in a.shape))
    row = lambda n: pl.BlockSpec((1, n), lambda b: (0, 0))
    return pl.pallas_call(
        _net_kernel,
        grid=(nb,),
        in_specs=[
            pl.BlockSpec((BB, 60, 60), lambda b: (b, 0, 0)),
            full(t1e), full(t1o), row(168),
            full(g2e), full(g2o), row(192),
            full(wf1_perm), row(120), full(wf2), row(84), full(wf3), row(2),
        ],
        out_specs=pl.BlockSpec((BB, 2), lambda b: (b, 0)),
        out_shape=jax.ShapeDtypeStruct((B, 2), jnp.float32),
        scratch_shapes=[
            pltpu.VMEM((BB * 56, 128), jnp.float32),
            pltpu.VMEM((BB * 56, 128), jnp.float32),
            pltpu.VMEM((BB * 14, 128), jnp.float32),
            pltpu.VMEM((BB * 14, 128), jnp.float32),
        ],
        compiler_params=pltpu.CompilerParams(
            dimension_semantics=("parallel",)),
    )(x.reshape(B, 60, 60), t1e, t1o, b1_row,
      g2e, g2o, b2_row,
      wf1_perm.astype(jnp.bfloat16), bf1.reshape(1, 120),
      wf2.astype(jnp.bfloat16), bf2.reshape(1, 84),
      wf3.astype(jnp.bfloat16), bf3.reshape(1, 2))
